# baseline (speedup 1.0000x reference)
import functools

import jax
import jax.numpy as jnp
from jax import lax
from jax.experimental import pallas as pl
from jax.experimental.pallas import tpu as pltpu
from jax.experimental.pallas import tpu_sc as plsc

D = 128
NN = 4096
NP = 3
LANES = 16
NC = 2
NS = 16
NW = NC * NS
CHUNKS = 2
RPW = NN // (NW * CHUNKS)
NCH = NP * CHUNKS
NG = RPW // LANES
MC = D // LANES
TS = LANES + 1
DIAG = RPW * TS
PW = D + LANES


def _tree_sum(xs):
  xs = list(xs)
  while len(xs) > 1:
    nxt = [xs[i] + xs[i + 1] for i in range(0, len(xs) - 1, 2)]
    if len(xs) % 2:
      nxt.append(xs[-1])
    xs = nxt
  return xs[0]


def _sc_partials(task, idx0, idx1, idx2, emb, w0, w1, w2):
  mesh = plsc.VectorSubcoreMesh(
      core_axis_name="c", subcore_axis_name="s", num_cores=NC)
  idx_is_pairs = idx0.ndim == 2

  @functools.partial(
      pl.kernel,
      out_type=jax.ShapeDtypeStruct((NW, NP, PW), jnp.float32),
      mesh=mesh,
      compiler_params=pltpu.CompilerParams(needs_layout_passes=False),
      scratch_types=[
          pltpu.VMEM((NCH, RPW), jnp.int32),
          pltpu.VMEM((RPW, 2), jnp.int32),
          pltpu.VMEM((RPW, D), jnp.float32),
          pltpu.VMEM((RPW, D), jnp.float32),
          pltpu.VMEM((D,), jnp.float32),
          pltpu.VMEM((NP, 1, 2 * D), jnp.float32),
          pltpu.VMEM((DIAG + LANES * TS,), jnp.float32),
          pltpu.VMEM((NP, PW), jnp.float32),
          pltpu.SemaphoreType.DMA,
          pltpu.SemaphoreType.DMA,
          pltpu.SemaphoreType.DMA,
      ] + [pltpu.SemaphoreType.DMA] * NCH,
  )
  def sc_kernel(task_hbm, i0, i1, i2, emb_hbm, w0_hbm, w1_hbm, w2_hbm,
                part_out, idx_v, pair_v, rows_a, rows_b, task_v, w_v, tbuf,
                acc_v, sem_a, sem_b, sem_w, *isems):
    cid = lax.axis_index("c")
    sid = lax.axis_index("s")
    wid = sid * NC + cid

    jv = lax.iota(jnp.int32, LANES)
    jv17 = jv * TS

    srcs = tuple((i0, i1, i2)[c // CHUNKS] for c in range(NCH))
    idx_waits = [None] * NCH
    for c in range(NCH):
      base = wid * (RPW * CHUNKS) + (c % CHUNKS) * RPW
      i_hbm = srcs[c]
      if idx_is_pairs:
        pltpu.sync_copy(i_hbm.at[pl.ds(base, RPW)], pair_v)
        zz = jv * 0
        for g in range(NG):
          rows16 = g * LANES + jv
          lo = plsc.load_gather(pair_v, [rows16, zz])
          hi = plsc.load_gather(pair_v, [rows16, zz + 1])
          idx_v[c, pl.ds(g * LANES, LANES)] = lo | hi
      else:
        idx_waits[c] = (
            pltpu.async_copy(i_hbm.at[pl.ds(base, RPW)], idx_v.at[c],
                             isems[c]))

    tw_waits = [pltpu.async_copy(task_hbm, task_v, sem_w)]
    for p, w_hbm in enumerate((w0_hbm, w1_hbm, w2_hbm)):
      tw_waits.append(pltpu.async_copy(w_hbm, w_v.at[p], sem_w))

    gathers = [None] * NCH
    if idx_waits[0] is not None:
      idx_waits[0].wait()
    gathers[0] = pltpu.async_copy(emb_hbm.at[idx_v.at[0]], rows_a, sem_a)
    if NCH > 1:
      if idx_waits[1] is not None:
        idx_waits[1].wait()
      gathers[1] = pltpu.async_copy(emb_hbm.at[idx_v.at[1]], rows_b, sem_b)

    for h in tw_waits:
      h.wait()

    cvecs = []
    for p in range(NP):
      t = _tree_sum([task_v[pl.ds(m * LANES, LANES)]
                     * w_v[p, 0, pl.ds(m * LANES, LANES)] for m in range(MC)])
      plsc.store_scatter(tbuf, [jv17 + DIAG], t)
      cols = [plsc.load_gather(tbuf, [((jv + k) & (LANES - 1)) * TS + DIAG])
              for k in range(LANES)]
      cvecs.append(_tree_sum(cols))

    bufs = (rows_a, rows_b)
    sems = (sem_a, sem_b)
    zero_init = (tuple(jnp.zeros((LANES,), jnp.float32) for _ in range(MC)),
                 jnp.zeros((LANES,), jnp.float32))
    carry_in = zero_init
    for c in range(NCH):
      p = c // CHUNKS
      if c >= 1 and c + 1 < NCH:
        if idx_waits[c + 1] is not None:
          idx_waits[c + 1].wait()
        gathers[c + 1] = pltpu.async_copy(
            emb_hbm.at[idx_v.at[c + 1]], bufs[(c + 1) % 2], sems[(c + 1) % 2])
      gathers[c].wait()
      rows_v = bufs[c % 2]
      cvec = cvecs[p]
      wch = [w_v[p, 0, pl.ds(D + m * LANES, LANES)] for m in range(MC)]

      init = carry_in

      @plsc.parallel_loop(0, NG, carry=init, unroll=2)
      def gloop(g, carry, rows_v=rows_v, cvec=cvec, wch=wch):
        accs, zacc = carry
        gb = g * LANES
        for j in range(LANES):
          row = gb + j
          t = _tree_sum([rows_v[row, pl.ds(m * LANES, LANES)] * wch[m]
                         for m in range(MC)])
          plsc.store_scatter(tbuf, [jv + row * TS], t)
        gbase = g * (LANES * TS)
        cols = [plsc.load_gather(tbuf, [jv17 + (gbase + k)])
                for k in range(LANES)]
        s = _tree_sum(cols)
        sig = 1.0 / (1.0 + jnp.exp(-(cvec + s)))
        e = jnp.exp(sig)
        zacc = zacc + e
        new_accs = list(accs)
        for j in range(LANES):
          ej = e[j]
          for m in range(MC):
            new_accs[m] = (
                new_accs[m] + ej * rows_v[gb + j, pl.ds(m * LANES, LANES)])
        return tuple(new_accs), zacc

      accs, zacc = gloop
      if c % CHUNKS == CHUNKS - 1:
        for m in range(MC):
          acc_v[p, pl.ds(m * LANES, LANES)] = accs[m]
        acc_v[p, pl.ds(D, LANES)] = zacc
        carry_in = zero_init
      else:
        carry_in = (accs, zacc)

    pltpu.sync_copy(acc_v, part_out.at[wid])

  return sc_kernel(task, idx0, idx1, idx2, emb, w0, w1, w2)


def _tc_finish(part, task2d, w1, w2, b2d, v):
  def body(part_ref, task_ref, w1_ref, w2_ref, b_ref, v_ref, out_ref):
    part = part_ref[...]
    red = jnp.sum(part, axis=0)
    paths_raw = red[:, :D]
    zden = jnp.sum(red[:, D:], axis=1, keepdims=True)
    paths = paths_raw / zden
    q = jnp.dot(task_ref[...], w1_ref[...])
    t = jnp.tanh(q + jnp.dot(paths, w2_ref[...]) + b_ref[...])
    logits = jnp.dot(t, v_ref[...])
    e = jnp.exp(logits - jnp.max(logits))
    sw = e / jnp.sum(e)
    out_ref[...] = jnp.sum(paths * sw, axis=0, keepdims=True)

  return pl.pallas_call(
      body,
      out_shape=jax.ShapeDtypeStruct((1, D), jnp.float32),
  )(part, task2d, w1, w2, b2d, v)


def _as_sc_idx(x):
  if x.dtype == jnp.int32:
    return x
  if x.dtype == jnp.int64:
    return jax.lax.bitcast_convert_type(x, jnp.int32)
  return x.astype(jnp.int32)


def kernel(task_repre, neighbors_p0, neighbors_p1, neighbors_p2, emb_table,
           W_p0, W_p1, W_p2, w1, w2, b, v):
  i0 = _as_sc_idx(neighbors_p0)
  i1 = _as_sc_idx(neighbors_p1)
  i2 = _as_sc_idx(neighbors_p2)
  part = _sc_partials(task_repre, i0, i1, i2, emb_table, W_p0, W_p1, W_p2)
  out = _tc_finish(part, task_repre.reshape(1, D), w1, w2,
                   b.reshape(1, D), v)
  return out.reshape(D)


# speedup vs baseline: 1.1210x; 1.1210x over previous
import functools

import jax
import jax.numpy as jnp
from jax import lax
from jax.experimental import pallas as pl
from jax.experimental.pallas import tpu as pltpu
from jax.experimental.pallas import tpu_sc as plsc

D = 128
NN = 4096
NP = 3
LANES = 16
NC = 2
NS = 16
NW = NC * NS
CHUNKS = 1
RPW = NN // (NW * CHUNKS)
NCH = NP * CHUNKS
NG = RPW // LANES
MC = D // LANES
TS = LANES + 1
DIAG = RPW * TS
PW = D + LANES


def _tree_sum(xs):
  xs = list(xs)
  while len(xs) > 1:
    nxt = [xs[i] + xs[i + 1] for i in range(0, len(xs) - 1, 2)]
    if len(xs) % 2:
      nxt.append(xs[-1])
    xs = nxt
  return xs[0]


def _sc_partials(task, idx0, idx1, idx2, emb, w0, w1, w2):
  mesh = plsc.VectorSubcoreMesh(
      core_axis_name="c", subcore_axis_name="s", num_cores=NC)
  idx_is_pairs = idx0.ndim == 2

  @functools.partial(
      pl.kernel,
      out_type=jax.ShapeDtypeStruct((NW, NP, PW), jnp.float32),
      mesh=mesh,
      compiler_params=pltpu.CompilerParams(needs_layout_passes=False),
      scratch_types=[
          pltpu.VMEM((NCH, RPW), jnp.int32),
          pltpu.VMEM((RPW, 2), jnp.int32),
          pltpu.VMEM((RPW, D), jnp.float32),
          pltpu.VMEM((RPW, D), jnp.float32),
          pltpu.VMEM((RPW, D), jnp.float32),
          pltpu.VMEM((D,), jnp.float32),
          pltpu.VMEM((NP, 1, 2 * D), jnp.float32),
          pltpu.VMEM((DIAG + LANES * TS,), jnp.float32),
          pltpu.VMEM((NP, PW), jnp.float32),
          pltpu.SemaphoreType.DMA,
          pltpu.SemaphoreType.DMA,
          pltpu.SemaphoreType.DMA,
          pltpu.SemaphoreType.DMA,
      ] + [pltpu.SemaphoreType.DMA] * NCH,
  )
  def sc_kernel(task_hbm, i0, i1, i2, emb_hbm, w0_hbm, w1_hbm, w2_hbm,
                part_out, idx_v, pair_v, rows_a, rows_b, rows_c, task_v, w_v,
                tbuf, acc_v, sem_a, sem_b, sem_c, sem_w, *isems):
    cid = lax.axis_index("c")
    sid = lax.axis_index("s")
    wid = sid * NC + cid

    jv = lax.iota(jnp.int32, LANES)
    jv17 = jv * TS

    srcs = tuple((i0, i1, i2)[c // CHUNKS] for c in range(NCH))
    idx_waits = [None] * NCH
    for c in range(NCH):
      base = wid * (RPW * CHUNKS) + (c % CHUNKS) * RPW
      i_hbm = srcs[c]
      if idx_is_pairs:
        pltpu.sync_copy(i_hbm.at[pl.ds(base, RPW)], pair_v)
        zz = jv * 0
        for g in range(NG):
          rows16 = g * LANES + jv
          lo = plsc.load_gather(pair_v, [rows16, zz])
          hi = plsc.load_gather(pair_v, [rows16, zz + 1])
          idx_v[c, pl.ds(g * LANES, LANES)] = lo | hi
      else:
        idx_waits[c] = (
            pltpu.async_copy(i_hbm.at[pl.ds(base, RPW)], idx_v.at[c],
                             isems[c]))

    tw_waits = [pltpu.async_copy(task_hbm, task_v, sem_w)]
    for p, w_hbm in enumerate((w0_hbm, w1_hbm, w2_hbm)):
      tw_waits.append(pltpu.async_copy(w_hbm, w_v.at[p], sem_w))

    bufs = (rows_a, rows_b, rows_c)
    sems = (sem_a, sem_b, sem_c)
    nbuf = min(len(bufs), NCH)
    gathers = [None] * NCH
    for c in range(nbuf):
      if idx_waits[c] is not None:
        idx_waits[c].wait()
      gathers[c] = pltpu.async_copy(
          emb_hbm.at[idx_v.at[c]], bufs[c % len(bufs)], sems[c % len(sems)])

    for h in tw_waits:
      h.wait()

    cvecs = []
    for p in range(NP):
      t = _tree_sum([task_v[pl.ds(m * LANES, LANES)]
                     * w_v[p, 0, pl.ds(m * LANES, LANES)] for m in range(MC)])
      plsc.store_scatter(tbuf, [jv17 + DIAG], t)
      cols = [plsc.load_gather(tbuf, [((jv + k) & (LANES - 1)) * TS + DIAG])
              for k in range(LANES)]
      cvecs.append(_tree_sum(cols))

    zero_init = (tuple(jnp.zeros((LANES,), jnp.float32) for _ in range(MC)),
                 jnp.zeros((LANES,), jnp.float32))
    carry_in = zero_init
    for c in range(NCH):
      p = c // CHUNKS
      gathers[c].wait()
      rows_v = bufs[c % len(bufs)]
      cvec = cvecs[p]
      wch = [w_v[p, 0, pl.ds(D + m * LANES, LANES)] for m in range(MC)]

      init = carry_in

      @plsc.parallel_loop(0, NG, carry=init, unroll=2)
      def gloop(g, carry, rows_v=rows_v, cvec=cvec, wch=wch):
        accs, zacc = carry
        gb = g * LANES
        for j in range(LANES):
          row = gb + j
          t = _tree_sum([rows_v[row, pl.ds(m * LANES, LANES)] * wch[m]
                         for m in range(MC)])
          plsc.store_scatter(tbuf, [jv + row * TS], t)
        gbase = g * (LANES * TS)
        cols = [plsc.load_gather(tbuf, [jv17 + (gbase + k)])
                for k in range(LANES)]
        s = _tree_sum(cols)
        sig = 1.0 / (1.0 + jnp.exp(-(cvec + s)))
        e = jnp.exp(sig)
        zacc = zacc + e
        new_accs = list(accs)
        for j in range(LANES):
          ej = e[j]
          for m in range(MC):
            new_accs[m] = (
                new_accs[m] + ej * rows_v[gb + j, pl.ds(m * LANES, LANES)])
        return tuple(new_accs), zacc

      accs, zacc = gloop
      if c % CHUNKS == CHUNKS - 1:
        for m in range(MC):
          acc_v[p, pl.ds(m * LANES, LANES)] = accs[m]
        acc_v[p, pl.ds(D, LANES)] = zacc
        carry_in = zero_init
      else:
        carry_in = (accs, zacc)
      if c + nbuf < NCH:
        if idx_waits[c + nbuf] is not None:
          idx_waits[c + nbuf].wait()
        gathers[c + nbuf] = pltpu.async_copy(
            emb_hbm.at[idx_v.at[c + nbuf]], bufs[(c + nbuf) % len(bufs)],
            sems[(c + nbuf) % len(sems)])

    pltpu.sync_copy(acc_v, part_out.at[wid])

  return sc_kernel(task, idx0, idx1, idx2, emb, w0, w1, w2)


def _tc_finish(part, task2d, w1, w2, b2d, v):
  def body(part_ref, task_ref, w1_ref, w2_ref, b_ref, v_ref, out_ref):
    part = part_ref[...]
    red = jnp.sum(part, axis=0)
    paths_raw = red[:, :D]
    zden = jnp.sum(red[:, D:], axis=1, keepdims=True)
    paths = paths_raw / zden
    q = jnp.dot(task_ref[...], w1_ref[...])
    t = jnp.tanh(q + jnp.dot(paths, w2_ref[...]) + b_ref[...])
    logits = jnp.dot(t, v_ref[...])
    e = jnp.exp(logits - jnp.max(logits))
    sw = e / jnp.sum(e)
    out_ref[...] = jnp.sum(paths * sw, axis=0, keepdims=True)

  return pl.pallas_call(
      body,
      out_shape=jax.ShapeDtypeStruct((1, D), jnp.float32),
  )(part, task2d, w1, w2, b2d, v)


def _as_sc_idx(x):
  if x.dtype == jnp.int32:
    return x
  if x.dtype == jnp.int64:
    return jax.lax.bitcast_convert_type(x, jnp.int32)
  return x.astype(jnp.int32)


def kernel(task_repre, neighbors_p0, neighbors_p1, neighbors_p2, emb_table,
           W_p0, W_p1, W_p2, w1, w2, b, v):
  i0 = _as_sc_idx(neighbors_p0)
  i1 = _as_sc_idx(neighbors_p1)
  i2 = _as_sc_idx(neighbors_p2)
  part = _sc_partials(task_repre, i0, i1, i2, emb_table, W_p0, W_p1, W_p2)
  out = _tc_finish(part, task_repre.reshape(1, D), w1, w2,
                   b.reshape(1, D), v)
  return out.reshape(D)


# speedup vs baseline: 1.1445x; 1.0210x over previous
import functools

import jax
import jax.numpy as jnp
from jax import lax
from jax.experimental import pallas as pl
from jax.experimental.pallas import tpu as pltpu
from jax.experimental.pallas import tpu_sc as plsc

D = 128
NN = 4096
NP = 3
LANES = 16
NC = 2
NS = 16
NW = NC * NS
CHUNKS = 1
RPW = NN // (NW * CHUNKS)
NCH = NP * CHUNKS
NG = RPW // LANES
MC = D // LANES
TS = LANES + 1
DIAG = RPW * TS
PW = D + LANES


def _tree_sum(xs):
  xs = list(xs)
  while len(xs) > 1:
    nxt = [xs[i] + xs[i + 1] for i in range(0, len(xs) - 1, 2)]
    if len(xs) % 2:
      nxt.append(xs[-1])
    xs = nxt
  return xs[0]


def _sc_partials(task, idx0, idx1, idx2, emb, w0, w1, w2):
  mesh = plsc.VectorSubcoreMesh(
      core_axis_name="c", subcore_axis_name="s", num_cores=NC)
  idx_is_pairs = idx0.ndim == 2

  @functools.partial(
      pl.kernel,
      out_type=jax.ShapeDtypeStruct((NW, NP, PW), jnp.float32),
      mesh=mesh,
      compiler_params=pltpu.CompilerParams(needs_layout_passes=False),
      scratch_types=[
          pltpu.VMEM((NCH, RPW), jnp.int32),
          pltpu.VMEM((RPW, 2), jnp.int32),
          pltpu.VMEM((RPW, D), jnp.float32),
          pltpu.VMEM((RPW, D), jnp.float32),
          pltpu.VMEM((D,), jnp.float32),
          pltpu.VMEM((NP, 1, 2 * D), jnp.float32),
          pltpu.VMEM((DIAG + LANES * TS,), jnp.float32),
          pltpu.VMEM((NP, PW), jnp.float32),
          pltpu.SemaphoreType.DMA,
          pltpu.SemaphoreType.DMA,
          pltpu.SemaphoreType.DMA,
      ] + [pltpu.SemaphoreType.DMA] * NCH,
  )
  def sc_kernel(task_hbm, i0, i1, i2, emb_hbm, w0_hbm, w1_hbm, w2_hbm,
                part_out, idx_v, pair_v, rows_a, rows_b, task_v, w_v,
                tbuf, acc_v, sem_a, sem_b, sem_w, *isems):
    cid = lax.axis_index("c")
    sid = lax.axis_index("s")
    wid = sid * NC + cid

    jv = lax.iota(jnp.int32, LANES)
    jv17 = jv * TS

    srcs = tuple((i0, i1, i2)[c // CHUNKS] for c in range(NCH))
    idx_waits = [None] * NCH
    for c in range(NCH):
      base = wid * (RPW * CHUNKS) + (c % CHUNKS) * RPW
      i_hbm = srcs[c]
      if idx_is_pairs:
        pltpu.sync_copy(i_hbm.at[pl.ds(base, RPW)], pair_v)
        zz = jv * 0
        for g in range(NG):
          rows16 = g * LANES + jv
          lo = plsc.load_gather(pair_v, [rows16, zz])
          hi = plsc.load_gather(pair_v, [rows16, zz + 1])
          idx_v[c, pl.ds(g * LANES, LANES)] = lo | hi
      else:
        idx_waits[c] = (
            pltpu.async_copy(i_hbm.at[pl.ds(base, RPW)], idx_v.at[c],
                             isems[c]))

    tw_waits = [pltpu.async_copy(task_hbm, task_v, sem_w)]
    for p, w_hbm in enumerate((w0_hbm, w1_hbm, w2_hbm)):
      tw_waits.append(pltpu.async_copy(w_hbm, w_v.at[p], sem_w))

    bufs = (rows_a, rows_b)
    sems = (sem_a, sem_b)
    nbuf = min(len(bufs), NCH)
    gathers = [None] * NCH
    for c in range(nbuf):
      if idx_waits[c] is not None:
        idx_waits[c].wait()
      gathers[c] = pltpu.async_copy(
          emb_hbm.at[idx_v.at[c]], bufs[c % len(bufs)], sems[c % len(sems)])

    for h in tw_waits:
      h.wait()

    cvecs = []
    for p in range(NP):
      t = _tree_sum([task_v[pl.ds(m * LANES, LANES)]
                     * w_v[p, 0, pl.ds(m * LANES, LANES)] for m in range(MC)])
      plsc.store_scatter(tbuf, [jv17 + DIAG], t)
      cols = [plsc.load_gather(tbuf, [((jv + k) & (LANES - 1)) * TS + DIAG])
              for k in range(LANES)]
      cvecs.append(_tree_sum(cols))

    zero_init = (tuple(jnp.zeros((LANES,), jnp.float32) for _ in range(MC)),
                 jnp.zeros((LANES,), jnp.float32))
    carry_in = zero_init
    for c in range(NCH):
      p = c // CHUNKS
      gathers[c].wait()
      rows_v = bufs[c % len(bufs)]
      cvec = cvecs[p]
      wch = [w_v[p, 0, pl.ds(D + m * LANES, LANES)] for m in range(MC)]

      init = carry_in

      @plsc.parallel_loop(0, NG, carry=init, unroll=2)
      def gloop(g, carry, rows_v=rows_v, cvec=cvec, wch=wch):
        accs, zacc = carry
        gb = g * LANES
        for j in range(LANES):
          row = gb + j
          t = _tree_sum([rows_v[row, pl.ds(m * LANES, LANES)] * wch[m]
                         for m in range(MC)])
          plsc.store_scatter(tbuf, [jv + row * TS], t)
        gbase = g * (LANES * TS)
        cols = [plsc.load_gather(tbuf, [jv17 + (gbase + k)])
                for k in range(LANES)]
        s = _tree_sum(cols)
        sig = 1.0 / (1.0 + jnp.exp(-(cvec + s)))
        e = jnp.exp(sig)
        zacc = zacc + e
        new_accs = list(accs)
        for j in range(LANES):
          ej = e[j]
          for m in range(MC):
            new_accs[m] = (
                new_accs[m] + ej * rows_v[gb + j, pl.ds(m * LANES, LANES)])
        return tuple(new_accs), zacc

      accs, zacc = gloop
      if c % CHUNKS == CHUNKS - 1:
        for m in range(MC):
          acc_v[p, pl.ds(m * LANES, LANES)] = accs[m]
        acc_v[p, pl.ds(D, LANES)] = zacc
        carry_in = zero_init
      else:
        carry_in = (accs, zacc)
      if c + nbuf < NCH:
        if idx_waits[c + nbuf] is not None:
          idx_waits[c + nbuf].wait()
        gathers[c + nbuf] = pltpu.async_copy(
            emb_hbm.at[idx_v.at[c + nbuf]], bufs[(c + nbuf) % len(bufs)],
            sems[(c + nbuf) % len(sems)])

    pltpu.sync_copy(acc_v, part_out.at[wid])

  return sc_kernel(task, idx0, idx1, idx2, emb, w0, w1, w2)


def _tc_finish(part, task2d, w1, w2, b2d, v):
  def body(part_ref, task_ref, w1_ref, w2_ref, b_ref, v_ref, out_ref):
    part = part_ref[...]
    red = jnp.sum(part, axis=0)
    paths_raw = red[:, :D]
    zden = jnp.sum(red[:, D:], axis=1, keepdims=True)
    paths = paths_raw / zden
    q = jnp.dot(task_ref[...], w1_ref[...])
    t = jnp.tanh(q + jnp.dot(paths, w2_ref[...]) + b_ref[...])
    logits = jnp.dot(t, v_ref[...])
    e = jnp.exp(logits - jnp.max(logits))
    sw = e / jnp.sum(e)
    out_ref[...] = jnp.sum(paths * sw, axis=0, keepdims=True)

  return pl.pallas_call(
      body,
      out_shape=jax.ShapeDtypeStruct((1, D), jnp.float32),
  )(part, task2d, w1, w2, b2d, v)


def _as_sc_idx(x):
  if x.dtype == jnp.int32:
    return x
  if x.dtype == jnp.int64:
    return jax.lax.bitcast_convert_type(x, jnp.int32)
  return x.astype(jnp.int32)


def kernel(task_repre, neighbors_p0, neighbors_p1, neighbors_p2, emb_table,
           W_p0, W_p1, W_p2, w1, w2, b, v):
  i0 = _as_sc_idx(neighbors_p0)
  i1 = _as_sc_idx(neighbors_p1)
  i2 = _as_sc_idx(neighbors_p2)
  part = _sc_partials(task_repre, i0, i1, i2, emb_table, W_p0, W_p1, W_p2)
  out = _tc_finish(part, task_repre.reshape(1, D), w1, w2,
                   b.reshape(1, D), v)
  return out.reshape(D)
